# baseline (speedup 1.0000x reference)
import functools

import jax
import jax.numpy as jnp
from jax import lax
from jax.experimental import pallas as pl
from jax.experimental.pallas import tpu as pltpu
from jax.experimental.pallas import tpu_sc as plsc

_NC = 2
_NS = 16
_NW = _NC * _NS
_WIN = 128


def _make_sc_gather(tok, d_e):
    per_w = tok // _NW
    n_win = per_w // _WIN
    mesh = plsc.VectorSubcoreMesh(core_axis_name="c", subcore_axis_name="s")

    @functools.partial(
        pl.kernel,
        mesh=mesh,
        compiler_params=pltpu.CompilerParams(use_tc_tiling_on_sc=False),
        out_type=jax.ShapeDtypeStruct((tok, d_e), jnp.bfloat16),
        scratch_types=[
            pltpu.VMEM((per_w,), jnp.int32),
            pltpu.VMEM((_WIN, d_e), jnp.bfloat16),
            pltpu.VMEM((_WIN, d_e), jnp.bfloat16),
            pltpu.SemaphoreType.DMA,
            pltpu.SemaphoreType.DMA,
        ],
        cost_estimate=pl.CostEstimate(
            flops=0,
            transcendentals=0,
            bytes_accessed=2 * tok * d_e * 4 + tok * 4,
        ),
    )
    def gather_k(ids_hbm, table_hbm, out_hbm, idx_v, buf0, buf1, sem0, sem1):
        wid = lax.axis_index("s") * _NC + lax.axis_index("c")
        base = wid * per_w
        pltpu.sync_copy(ids_hbm.at[pl.ds(base, per_w)], idx_v)

        def gather_win(w, buf, sem):
            pltpu.async_copy(
                table_hbm.at[idx_v.at[pl.ds(w * _WIN, _WIN)]], buf, sem
            )

        def drain(buf, sem):
            pltpu.make_async_copy(
                table_hbm.at[idx_v.at[pl.ds(0, _WIN)]], buf, sem
            ).wait()

        gather_win(0, buf0, sem0)

        def body(i, carry):
            w0 = 2 * i
            gather_win(w0 + 1, buf1, sem1)
            drain(buf0, sem0)
            pltpu.sync_copy(buf0, out_hbm.at[pl.ds(base + w0 * _WIN, _WIN)])

            @pl.when(i < n_win // 2 - 1)
            def _():
                gather_win(w0 + 2, buf0, sem0)

            drain(buf1, sem1)
            pltpu.sync_copy(
                buf1, out_hbm.at[pl.ds(base + (w0 + 1) * _WIN, _WIN)]
            )
            return carry

        lax.fori_loop(0, n_win // 2, body, 0)

    return gather_k


def _tc_fused(x3, pos_col, w2, b2, bt, half, ub):
    rows = half * 128

    def body(x_ref, pos_ref, w_ref, b_ref, et_ref, ht_ref):
        for u in range(ub):
            slab_t = jnp.transpose(x_ref[u].astype(jnp.float32))
            et = slab_t + pos_ref[u * 128:(u + 1) * 128, :]
            et_ref[u * 128:(u + 1) * 128, :] = et
            z = jnp.dot(w_ref[...], et, preferred_element_type=jnp.float32)
            ht = jax.nn.sigmoid(z + b_ref[...])
            y = jnp.swapaxes(
                ht.reshape(32, 4, bt // 128, 128), 1, 2
            ).reshape(bt, 128)
            ht_ref[u * bt:(u + 1) * bt, :] = y

    grid = (half // ub,)
    return pl.pallas_call(
        body,
        grid=grid,
        in_specs=[
            pl.BlockSpec((ub, bt, 128), lambda i: (i, 0, 0)),
            pl.BlockSpec((ub * 128, 1), lambda i: (i, 0)),
            pl.BlockSpec((128, 128), lambda i: (0, 0)),
            pl.BlockSpec((128, 1), lambda i: (0, 0)),
        ],
        out_specs=[
            pl.BlockSpec((ub * 128, bt), lambda i: (i, 0)),
            pl.BlockSpec((ub * bt, 128), lambda i: (i, 0)),
        ],
        out_shape=[
            jax.ShapeDtypeStruct((rows, bt), jnp.float32),
            jax.ShapeDtypeStruct((half * bt, 128), jnp.float32),
        ],
    )(x3, pos_col, w2, b2)


def kernel(token_ids, token_embed, pos_embed, W_h, b_h):
    bt, st = token_ids.shape
    vocab, d_e = token_embed.shape
    nb = W_h.shape[1] // 4
    tok = bt * st
    half = st // 2

    ids_perm = jnp.transpose(
        token_ids.astype(jnp.int32).reshape(bt, half, 2), (1, 0, 2)
    ).reshape(tok)

    tbl_pairs = jax.lax.optimization_barrier(
        token_embed.astype(jnp.bfloat16).reshape(vocab // 2, 2 * d_e)
    )
    tbl_lin = tbl_pairs.reshape(vocab, d_e)

    e_raw = _make_sc_gather(tok, d_e)(ids_perm, tbl_lin)
    x3 = e_raw.reshape(half, bt, 2 * d_e)

    pos_col = pos_embed[:st].reshape(st * d_e, 1)
    w_t = W_h.T
    w2 = (
        jnp.zeros((2 * d_e, 2 * d_e), jnp.float32)
        .at[:d_e, :d_e].set(w_t)
        .at[d_e:, d_e:].set(w_t)
    )
    b2 = jnp.concatenate([b_h, b_h]).reshape(2 * d_e, 1)

    e_t, h_y = _tc_fused(x3, pos_col, w2, b2, bt, half, ub=10)

    e = jnp.transpose(e_t.reshape(st, d_e, bt), (2, 0, 1))
    h_t = jnp.swapaxes(
        h_y.reshape(st * d_e // 4, bt // 128, 4, 128), 1, 2
    ).reshape(st * d_e, bt)
    h = jnp.transpose(h_t.reshape(st, nb, 4, bt), (3, 0, 1, 2))
    wm = jnp.zeros((bt, st, nb, nb), jnp.float32)
    return (e, h, wm)


# speedup vs baseline: 1.4797x; 1.4797x over previous
import functools

import jax
import jax.numpy as jnp
from jax import lax
from jax.experimental import pallas as pl
from jax.experimental.pallas import tpu as pltpu
from jax.experimental.pallas import tpu_sc as plsc

_NC = 2
_NS = 16
_NW = _NC * _NS
_WIN = 128


def _make_sc_gather(tok, d_e):
    per_w = tok // _NW
    n_win = per_w // _WIN
    mesh = plsc.VectorSubcoreMesh(core_axis_name="c", subcore_axis_name="s")

    @functools.partial(
        pl.kernel,
        mesh=mesh,
        compiler_params=pltpu.CompilerParams(use_tc_tiling_on_sc=False),
        out_type=jax.ShapeDtypeStruct((tok, d_e), jnp.float32),
        scratch_types=[
            pltpu.VMEM((per_w,), jnp.int32),
            pltpu.VMEM((_WIN, d_e), jnp.float32),
            pltpu.VMEM((_WIN, d_e), jnp.float32),
            pltpu.SemaphoreType.DMA,
            pltpu.SemaphoreType.DMA,
        ],
        cost_estimate=pl.CostEstimate(
            flops=0,
            transcendentals=0,
            bytes_accessed=2 * tok * d_e * 4 + tok * 4,
        ),
    )
    def gather_k(ids_hbm, table_hbm, out_hbm, idx_v, buf0, buf1, sem0, sem1):
        wid = lax.axis_index("s") * _NC + lax.axis_index("c")
        base = wid * per_w
        pltpu.sync_copy(ids_hbm.at[pl.ds(base, per_w)], idx_v)

        def gather_win(w, buf, sem):
            pltpu.async_copy(
                table_hbm.at[idx_v.at[pl.ds(w * _WIN, _WIN)]], buf, sem
            )

        def drain(buf, sem):
            pltpu.make_async_copy(
                table_hbm.at[idx_v.at[pl.ds(0, _WIN)]], buf, sem
            ).wait()

        gather_win(0, buf0, sem0)

        def body(i, carry):
            w0 = 2 * i
            gather_win(w0 + 1, buf1, sem1)
            drain(buf0, sem0)
            pltpu.sync_copy(buf0, out_hbm.at[pl.ds(base + w0 * _WIN, _WIN)])

            @pl.when(i < n_win // 2 - 1)
            def _():
                gather_win(w0 + 2, buf0, sem0)

            drain(buf1, sem1)
            pltpu.sync_copy(
                buf1, out_hbm.at[pl.ds(base + (w0 + 1) * _WIN, _WIN)]
            )
            return carry

        lax.fori_loop(0, n_win // 2, body, 0)

    return gather_k


def _tc_fused(x3, pos_col, w2, b2, bt, half, ub):
    rows = half * 128

    def body(x_ref, pos_ref, w_ref, b_ref, et_ref, ht_ref):
        for u in range(ub):
            slab_t = jnp.transpose(x_ref[u])
            et = slab_t + pos_ref[u * 128:(u + 1) * 128, :]
            et_ref[u * 128:(u + 1) * 128, :] = et
            z = jnp.dot(w_ref[...], et, preferred_element_type=jnp.float32)
            ht = jax.nn.sigmoid(z + b_ref[...])
            y = jnp.swapaxes(
                ht.reshape(32, 4, bt // 128, 128), 1, 2
            ).reshape(bt, 128)
            ht_ref[u * bt:(u + 1) * bt, :] = y

    grid = (half // ub,)
    return pl.pallas_call(
        body,
        grid=grid,
        in_specs=[
            pl.BlockSpec((ub, bt, 128), lambda i: (i, 0, 0)),
            pl.BlockSpec((ub * 128, 1), lambda i: (i, 0)),
            pl.BlockSpec((128, 128), lambda i: (0, 0)),
            pl.BlockSpec((128, 1), lambda i: (0, 0)),
        ],
        out_specs=[
            pl.BlockSpec((ub * 128, bt), lambda i: (i, 0)),
            pl.BlockSpec((ub * bt, 128), lambda i: (i, 0)),
        ],
        out_shape=[
            jax.ShapeDtypeStruct((rows, bt), jnp.float32),
            jax.ShapeDtypeStruct((half * bt, 128), jnp.float32),
        ],
    )(x3, pos_col, w2, b2)


def kernel(token_ids, token_embed, pos_embed, W_h, b_h):
    bt, st = token_ids.shape
    vocab, d_e = token_embed.shape
    nb = W_h.shape[1] // 4
    tok = bt * st
    half = st // 2

    ids_perm = jnp.transpose(
        token_ids.astype(jnp.int32).reshape(bt, half, 2), (1, 0, 2)
    ).reshape(tok)

    tbl_pairs = jax.lax.optimization_barrier(
        token_embed.reshape(vocab // 2, 2 * d_e)
    )
    tbl_lin = tbl_pairs.reshape(vocab, d_e)

    e_raw = _make_sc_gather(tok, d_e)(ids_perm, tbl_lin)
    x3 = e_raw.reshape(half, bt, 2 * d_e)

    pos_col = pos_embed[:st].reshape(st * d_e, 1)
    w_t = W_h.T
    w2 = (
        jnp.zeros((2 * d_e, 2 * d_e), jnp.float32)
        .at[:d_e, :d_e].set(w_t)
        .at[d_e:, d_e:].set(w_t)
    )
    b2 = jnp.concatenate([b_h, b_h]).reshape(2 * d_e, 1)

    e_t, h_y = _tc_fused(x3, pos_col, w2, b2, bt, half, ub=10)

    e = jnp.transpose(e_t.reshape(st, d_e, bt), (2, 0, 1))
    h_t = jnp.swapaxes(
        h_y.reshape(st * d_e // 4, bt // 128, 4, 128), 1, 2
    ).reshape(st * d_e, bt)
    h = jnp.transpose(h_t.reshape(st, nb, 4, bt), (3, 0, 1, 2))
    wm = jnp.zeros((bt, st, nb, nb), jnp.float32)
    return (e, h, wm)
